# baseline (speedup 1.0000x reference)
import functools

import jax
import jax.numpy as jnp
from jax import lax
from jax.experimental import pallas as pl
from jax.experimental.pallas import tpu as pltpu
from jax.experimental.pallas import tpu_sc as plsc

VOCAB = 100000
D = 128
GRID = 30
B = 1024
P = GRID * GRID
PPOS = 944
CH = 128
CPB = B // CH
NCHUNK = P * CPB
NW = 32
CPW = NCHUNK // NW
GRP = 45
NGRP = CPW // GRP
NBUF = 5
LOOKAHEAD = 2
PROWS = 40
LANES = 16
NVREG = D // LANES


def _pos_tc_body(row_ref, col_ref, out_ref):
    col = col_ref[...]
    for i in range(GRID):
        out_ref[pl.ds(GRID * i, GRID), :] = row_ref[i, :][None, :] + col
    out_ref[pl.ds(P, GRID), :] = col
    out_ref[pl.ds(P + GRID, PPOS - P - GRID), :] = col[: PPOS - P - GRID, :]


def _build_pos(row_table, col_table):
    return pl.pallas_call(
        _pos_tc_body,
        out_shape=jax.ShapeDtypeStruct((PPOS, D), jnp.float32),
    )(row_table, col_table)


def _sc_body(idx_hbm, table_hbm, pos_hbm, out_hbm, idx_v, pos_v, rows_v,
             sems_g, sems_w):
    c = lax.axis_index("c")
    s = lax.axis_index("s")
    w = s * 2 + c
    base_c = w * CPW

    palign = pl.multiple_of((base_c // CPB) // 8 * 8, 8)
    pltpu.sync_copy(pos_hbm.at[pl.ds(palign, PROWS)], pos_v)

    def group_body(g, carry):
        c0 = base_c + g * GRP
        pltpu.sync_copy(idx_hbm.at[pl.ds(c0 * CH, GRP * CH)], idx_v)

        gathers = {}
        writes = {}

        def start_gather(k):
            gathers[k] = pltpu.async_copy(
                table_hbm.at[idx_v.at[pl.ds(k * CH, CH)]],
                rows_v.at[k % NBUF], sems_g[k % NBUF])

        for k in range(LOOKAHEAD):
            start_gather(k)
        for k in range(GRP):
            gathers.pop(k).wait()
            ck = c0 + k
            prow = ck // CPB - palign
            pv = [pos_v[prow, pl.ds(v * LANES, LANES)] for v in range(NVREG)]

            def row_body(r2, _):
                for rr in range(2):
                    r = r2 * 2 + rr
                    for v in range(NVREG):
                        plsc.addupdate(
                            rows_v.at[k % NBUF, r, pl.ds(v * LANES, LANES)],
                            pv[v])
                return 0

            lax.fori_loop(0, CH // 2, row_body, 0)

            writes[k] = pltpu.async_copy(
                rows_v.at[k % NBUF], out_hbm.at[pl.ds(ck * CH, CH)],
                sems_w[k % NBUF])
            if k + LOOKAHEAD < GRP:
                prev = k + LOOKAHEAD - NBUF
                if prev >= 0:
                    writes.pop(prev).wait()
                start_gather(k + LOOKAHEAD)
        for k in sorted(writes):
            writes[k].wait()
        return carry

    lax.fori_loop(0, NGRP, group_body, 0)


def _sc_gather(idx1, token_table, pos):
    mesh = plsc.VectorSubcoreMesh(core_axis_name="c", subcore_axis_name="s")
    run = pl.kernel(
        lambda *refs: _sc_body(refs[0], refs[1], refs[2], refs[3],
                               refs[4], refs[5], refs[6],
                               list(refs[7:7 + NBUF]),
                               list(refs[7 + NBUF:7 + 2 * NBUF])),
        out_type=jax.ShapeDtypeStruct((P * B, D), jnp.float32),
        mesh=mesh,
        scratch_types=[
            pltpu.VMEM((GRP * CH,), jnp.int32),
            pltpu.VMEM((PROWS, D), jnp.float32),
            pltpu.VMEM((NBUF, CH, D), jnp.float32),
        ] + [pltpu.SemaphoreType.DMA] * (2 * NBUF),
    )
    return run(idx1, token_table, pos)


@jax.jit
def kernel(inputs, token_table, row_table, col_table):
    pos = _build_pos(row_table, col_table)
    idx1 = inputs.astype(jnp.int32).T.reshape(P * B)
    out = _sc_gather(idx1, token_table, pos)
    return out.reshape(P, B, D).transpose(1, 0, 2)
